# Initial kernel scaffold; baseline (speedup 1.0000x reference)
import jax
import jax.numpy as jnp
from jax.experimental import pallas as pl


def kernel(feats, edge_index, W0, W1, W2):
    raise NotImplementedError("write your pallas kernel here")


# speedup vs baseline: 5.4260x; 5.4260x over previous
import functools

import jax
import jax.numpy as jnp
from jax import lax
from jax.experimental import pallas as pl
from jax.experimental.pallas import tpu as pltpu
from jax.experimental.pallas import tpu_sc as plsc

NC = 2
NS = 16
CHUNK = 128


def _zero_f32(ref, n_rows, n_cols):
    @pl.loop(0, n_rows)
    def _(i):
        for j in range(n_cols // 16):
            ref[i, pl.ds(j * 16, 16)] = jnp.zeros((16,), jnp.float32)


def _strided_rows(s, n_rows, fn):
    n_full = n_rows // 128
    tail = n_rows % 128

    @pl.loop(s, n_full, step=NS)
    def _(p):
        fn(p * 128, 128)

    if tail:
        @pl.when(s == n_full % NS)
        def _():
            fn(n_full * 128, tail)


def _deg_body(dst_hbm, out_hbm, hist, ibuf, n_edges, n_nodes):
    c = lax.axis_index("c")
    s = lax.axis_index("s")
    wid = c * NS + s
    per_tile = n_edges // (NC * NS)

    @pl.loop(0, n_nodes // 16)
    def _(i):
        hist[pl.ds(i * 16, 16)] = jnp.zeros((16,), jnp.float32)

    pltpu.sync_copy(dst_hbm.at[pl.ds(wid * per_tile, per_tile)], ibuf)

    @pl.loop(0, per_tile // 16)
    def _(i):
        v = ibuf[pl.ds(i * 16, 16)]
        cnt, last = plsc.scan_count(v)
        plsc.addupdate_scatter(hist, [v], cnt.astype(jnp.float32), mask=last)

    pltpu.sync_copy(hist, out_hbm.at[wid])


def _hop_body(g_hbm, src_hbm, dst_hbm, out_hbm, acc, sidx, didx, rows, sem,
              n_edges, n_nodes):
    c = lax.axis_index("c")
    s = lax.axis_index("s")

    _zero_f32(rows, 128, 128)
    _strided_rows(s, n_nodes, lambda off, sz: pltpu.sync_copy(
        rows.at[pl.ds(0, sz)], acc.at[pl.ds(off, sz)]))
    plsc.subcore_barrier()

    n_chunks = n_edges // CHUNK
    half = n_chunks // NC
    lo = c * half + s

    @pl.loop(lo, (c + 1) * half, step=NS)
    def _(j):
        pltpu.sync_copy(src_hbm.at[pl.ds(j * CHUNK, CHUNK)], sidx)
        pltpu.sync_copy(dst_hbm.at[pl.ds(j * CHUNK, CHUNK)], didx)
        pltpu.async_copy(g_hbm.at[sidx], rows, sem).wait()
        pltpu.sync_copy(rows, acc.at[didx], add=True)

    plsc.subcore_barrier()
    _strided_rows(s, n_nodes, lambda off, sz: pltpu.sync_copy(
        acc.at[pl.ds(off, sz)], out_hbm.at[c, pl.ds(off, sz)]))


def _sc_deg(dst, n_nodes):
    n_edges = dst.shape[0]
    mesh = plsc.VectorSubcoreMesh(core_axis_name="c", subcore_axis_name="s")
    body = functools.partial(_deg_body, n_edges=n_edges, n_nodes=n_nodes)
    return pl.kernel(
        body,
        out_type=jax.ShapeDtypeStruct((NC * NS, n_nodes), jnp.float32),
        mesh=mesh,
        scratch_types=[
            pltpu.VMEM((n_nodes,), jnp.float32),
            pltpu.VMEM((n_edges // (NC * NS),), jnp.int32),
        ],
        compiler_params=pltpu.CompilerParams(needs_layout_passes=False),
    )(dst)


def _sc_hop(g, src, dst):
    n_nodes, d = g.shape
    n_edges = src.shape[0]
    mesh = plsc.VectorSubcoreMesh(core_axis_name="c", subcore_axis_name="s")
    body = functools.partial(_hop_body, n_edges=n_edges, n_nodes=n_nodes)
    return pl.kernel(
        body,
        out_type=jax.ShapeDtypeStruct((NC, n_nodes, d), jnp.float32),
        mesh=mesh,
        scratch_types=[
            pltpu.VMEM_SHARED((n_nodes, d), jnp.float32),
            pltpu.VMEM((CHUNK,), jnp.int32),
            pltpu.VMEM((CHUNK,), jnp.int32),
            pltpu.VMEM((CHUNK, d), jnp.float32),
            pltpu.SemaphoreType.DMA,
        ],
    )(g, src, dst)


_BLK = 1000


def _tc0_body(degp_ref, x_ref, w_ref, out_ref, g_ref, norm_ref):
    deg = jnp.sum(degp_ref[...], axis=1)[:, None]
    norm = lax.rsqrt(jnp.maximum(deg, 1.0))
    x = x_ref[...]
    out_ref[...] = lax.dot_general(
        x, w_ref[...], (((1,), (1,)), ((), ())),
        preferred_element_type=jnp.float32)
    g_ref[...] = x * norm
    norm_ref[...] = norm


def _tc_first(degp, feats, w):
    n, d = feats.shape
    grid = (n // _BLK,)
    return pl.pallas_call(
        _tc0_body,
        grid=grid,
        in_specs=[
            pl.BlockSpec((_BLK, NC * NS), lambda i: (i, 0)),
            pl.BlockSpec((_BLK, d), lambda i: (i, 0)),
            pl.BlockSpec((d, d), lambda i: (0, 0)),
        ],
        out_specs=[
            pl.BlockSpec((_BLK, d), lambda i: (i, 0)),
            pl.BlockSpec((_BLK, d), lambda i: (i, 0)),
            pl.BlockSpec((_BLK, 1), lambda i: (i, 0)),
        ],
        out_shape=[
            jax.ShapeDtypeStruct((n, d), jnp.float32),
            jax.ShapeDtypeStruct((n, d), jnp.float32),
            jax.ShapeDtypeStruct((n, 1), jnp.float32),
        ],
    )(degp, feats, w)


def _tc_hop_body(p_ref, norm_ref, w_ref, out_ref, g_ref):
    nrm = norm_ref[...]
    h = (p_ref[0] + p_ref[1]) * nrm
    out_ref[...] = lax.dot_general(
        h, w_ref[...], (((1,), (1,)), ((), ())),
        preferred_element_type=jnp.float32)
    if g_ref is not None:
        g_ref[...] = h * nrm


def _tc_hop(partials, norm, w, want_g):
    _, n, d = partials.shape
    grid = (n // _BLK,)
    out_shape = [jax.ShapeDtypeStruct((n, d), jnp.float32)]
    out_specs = [pl.BlockSpec((_BLK, d), lambda i: (i, 0))]
    if want_g:
        out_shape.append(jax.ShapeDtypeStruct((n, d), jnp.float32))
        out_specs.append(pl.BlockSpec((_BLK, d), lambda i: (i, 0)))
        body = _tc_hop_body
    else:
        def body(p_ref, norm_ref, w_ref, out_ref):
            _tc_hop_body(p_ref, norm_ref, w_ref, out_ref, None)
    res = pl.pallas_call(
        body,
        grid=grid,
        in_specs=[
            pl.BlockSpec((NC, _BLK, d), lambda i: (0, i, 0)),
            pl.BlockSpec((_BLK, 1), lambda i: (i, 0)),
            pl.BlockSpec((d, d), lambda i: (0, 0)),
        ],
        out_specs=out_specs,
        out_shape=out_shape,
    )(partials, norm, w)
    return res if want_g else (res[0], None)


def kernel(feats, edge_index, W0, W1, W2):
    n = feats.shape[0]
    src = edge_index[0]
    dst = edge_index[1]

    degp = _sc_deg(dst, n).T
    out0, g0, norm = _tc_first(degp, feats, W0)
    p1 = _sc_hop(g0, src, dst)
    out1, g1 = _tc_hop(p1, norm, W1, True)
    p2 = _sc_hop(g1, src, dst)
    out2, _ = _tc_hop(p2, norm, W2, False)
    return jnp.concatenate([out0, out1, out2], axis=1)


# speedup vs baseline: 8.8039x; 1.6225x over previous
import functools

import jax
import jax.numpy as jnp
from jax import lax
from jax.experimental import pallas as pl
from jax.experimental.pallas import tpu as pltpu
from jax.experimental.pallas import tpu_sc as plsc

NC = 2
NS = 16
CHUNK = 128


def _zero_f32(ref, n_rows, n_cols):
    @pl.loop(0, n_rows)
    def _(i):
        for j in range(n_cols // 16):
            ref[i, pl.ds(j * 16, 16)] = jnp.zeros((16,), jnp.float32)


def _strided_rows(s, n_rows, fn):
    n_full = n_rows // 128
    tail = n_rows % 128

    @pl.loop(s, n_full, step=NS)
    def _(p):
        fn(p * 128, 128)

    if tail:
        @pl.when(s == n_full % NS)
        def _():
            fn(n_full * 128, tail)


def _deg_body(dst_hbm, out_hbm, hist, ibuf, n_edges, n_nodes):
    c = lax.axis_index("c")
    s = lax.axis_index("s")
    wid = c * NS + s
    per_tile = n_edges // (NC * NS)

    @pl.loop(0, n_nodes // 16)
    def _(i):
        hist[pl.ds(i * 16, 16)] = jnp.zeros((16,), jnp.float32)

    pltpu.sync_copy(dst_hbm.at[pl.ds(wid * per_tile, per_tile)], ibuf)

    @pl.loop(0, per_tile // 16)
    def _(i):
        v = ibuf[pl.ds(i * 16, 16)]
        cnt, last = plsc.scan_count(v)
        plsc.addupdate_scatter(hist, [v], cnt.astype(jnp.float32), mask=last)

    pltpu.sync_copy(hist, out_hbm.at[wid])


NBUF = 2
IB = 40
JUNK = 64


def _hop_body(g_hbm, src_hbm, dst_hbm, out_hbm, acc, sidx, didx, rows,
              gsems, ssems, n_edges, n_nodes):
    c = lax.axis_index("c")
    s = lax.axis_index("s")

    _zero_f32(rows.at[0], 128, 128)
    _strided_rows(s, n_nodes, lambda off, sz: pltpu.sync_copy(
        rows.at[0, pl.ds(0, sz)], acc.at[pl.ds(off, sz)]))
    plsc.subcore_barrier()

    half = (n_edges // CHUNK) // NC
    n0 = half // NS
    nbatch = n0 // IB
    start = c * half + s * n0

    def load_batch(b):
        off = (start + b * IB) * CHUNK
        pltpu.sync_copy(src_hbm.at[pl.ds(off, IB * CHUNK)], sidx)
        pltpu.sync_copy(dst_hbm.at[pl.ds(off, IB * CHUNK)], didx)

    def start_gather(j, sl):
        pltpu.async_copy(g_hbm.at[sidx.at[pl.ds(j * CHUNK, CHUNK)]],
                         rows.at[sl], gsems[sl])

    def start_scatter(j, sl):
        pltpu.async_copy(rows.at[sl],
                         acc.at[didx.at[pl.ds(j * CHUNK, CHUNK)]],
                         ssems[sl], add=True)

    def wait_gather(j, sl):
        pltpu.make_async_copy(g_hbm.at[sidx.at[pl.ds(j * CHUNK, CHUNK)]],
                              rows.at[sl], gsems[sl]).wait()

    def wait_scatter(j, sl):
        pltpu.make_async_copy(rows.at[sl],
                              acc.at[didx.at[pl.ds(j * CHUNK, CHUNK)]],
                              ssems[sl]).wait()

    for b in range(nbatch):
        if b > 0:
            wait_scatter(IB - 1, (IB - 1) % NBUF)
        load_batch(b)
        start_gather(0, 0)
        for j in range(IB):
            sl = j % NBUF
            wait_gather(j, sl)
            start_scatter(j, sl)
            if j + 1 < IB:
                if j >= 1:
                    wait_scatter(j - 1, (j - 1) % NBUF)
                start_gather(j + 1, (j + 1) % NBUF)
    wait_scatter(IB - 1, (IB - 1) % NBUF)

    plsc.subcore_barrier()
    _strided_rows(s, n_nodes, lambda off, sz: pltpu.sync_copy(
        acc.at[pl.ds(off, sz)], out_hbm.at[c, pl.ds(off, sz)]))


def _sc_deg(dst, n_nodes):
    n_edges = dst.shape[0]
    mesh = plsc.VectorSubcoreMesh(core_axis_name="c", subcore_axis_name="s")
    body = functools.partial(_deg_body, n_edges=n_edges, n_nodes=n_nodes)
    return pl.kernel(
        body,
        out_type=jax.ShapeDtypeStruct((NC * NS, n_nodes), jnp.float32),
        mesh=mesh,
        scratch_types=[
            pltpu.VMEM((n_nodes,), jnp.float32),
            pltpu.VMEM((n_edges // (NC * NS),), jnp.int32),
        ],
        compiler_params=pltpu.CompilerParams(needs_layout_passes=False),
    )(dst)


def _sc_hop(g, src, dst):
    n_nodes, d = g.shape
    n_edges = src.shape[0]
    mesh = plsc.VectorSubcoreMesh(core_axis_name="c", subcore_axis_name="s")
    body = functools.partial(_hop_body, n_edges=n_edges, n_nodes=n_nodes)
    return pl.kernel(
        body,
        out_type=jax.ShapeDtypeStruct((NC, n_nodes, d), jnp.float32),
        mesh=mesh,
        scratch_types=[
            pltpu.VMEM_SHARED((n_nodes + JUNK, d), jnp.float32),
            pltpu.VMEM((IB * CHUNK,), jnp.int32),
            pltpu.VMEM((IB * CHUNK,), jnp.int32),
            pltpu.VMEM((NBUF, CHUNK, d), jnp.float32),
            [pltpu.SemaphoreType.DMA] * NBUF,
            [pltpu.SemaphoreType.DMA] * NBUF,
        ],
    )(g, src, dst)


_BLK = 1000


def _tc0_body(degp_ref, x_ref, w_ref, out_ref, g_ref, norm_ref):
    deg = jnp.sum(degp_ref[...], axis=1)[:, None]
    norm = lax.rsqrt(jnp.maximum(deg, 1.0))
    x = x_ref[...]
    out_ref[...] = lax.dot_general(
        x, w_ref[...], (((1,), (1,)), ((), ())),
        preferred_element_type=jnp.float32)
    g_ref[...] = x * norm
    norm_ref[...] = norm


def _tc_first(degp, feats, w):
    n, d = feats.shape
    grid = (n // _BLK,)
    return pl.pallas_call(
        _tc0_body,
        grid=grid,
        in_specs=[
            pl.BlockSpec((_BLK, NC * NS), lambda i: (i, 0)),
            pl.BlockSpec((_BLK, d), lambda i: (i, 0)),
            pl.BlockSpec((d, d), lambda i: (0, 0)),
        ],
        out_specs=[
            pl.BlockSpec((_BLK, d), lambda i: (i, 0)),
            pl.BlockSpec((_BLK, d), lambda i: (i, 0)),
            pl.BlockSpec((_BLK, 1), lambda i: (i, 0)),
        ],
        out_shape=[
            jax.ShapeDtypeStruct((n, d), jnp.float32),
            jax.ShapeDtypeStruct((n, d), jnp.float32),
            jax.ShapeDtypeStruct((n, 1), jnp.float32),
        ],
    )(degp, feats, w)


def _tc_hop_body(p_ref, norm_ref, w_ref, out_ref, g_ref):
    nrm = norm_ref[...]
    h = (p_ref[0] + p_ref[1]) * nrm
    out_ref[...] = lax.dot_general(
        h, w_ref[...], (((1,), (1,)), ((), ())),
        preferred_element_type=jnp.float32)
    if g_ref is not None:
        g_ref[...] = h * nrm


def _tc_hop(partials, norm, w, want_g):
    _, n, d = partials.shape
    grid = (n // _BLK,)
    out_shape = [jax.ShapeDtypeStruct((n, d), jnp.float32)]
    out_specs = [pl.BlockSpec((_BLK, d), lambda i: (i, 0))]
    if want_g:
        out_shape.append(jax.ShapeDtypeStruct((n, d), jnp.float32))
        out_specs.append(pl.BlockSpec((_BLK, d), lambda i: (i, 0)))
        body = _tc_hop_body
    else:
        def body(p_ref, norm_ref, w_ref, out_ref):
            _tc_hop_body(p_ref, norm_ref, w_ref, out_ref, None)
    res = pl.pallas_call(
        body,
        grid=grid,
        in_specs=[
            pl.BlockSpec((NC, _BLK, d), lambda i: (0, i, 0)),
            pl.BlockSpec((_BLK, 1), lambda i: (i, 0)),
            pl.BlockSpec((d, d), lambda i: (0, 0)),
        ],
        out_specs=out_specs,
        out_shape=out_shape,
    )(partials, norm, w)
    return res if want_g else (res[0], None)


def kernel(feats, edge_index, W0, W1, W2):
    n = feats.shape[0]
    src = edge_index[0]
    dst = edge_index[1]
    e = src.shape[0]

    unit = CHUNK * NC * NS * IB
    e_pad = -(-e // unit) * unit
    pad = e_pad - e
    if pad:
        fill = jnp.arange(pad, dtype=jnp.int32)
        src_p = jnp.concatenate([src, fill % n])
        dst_p = jnp.concatenate([dst, n + fill % JUNK])
    else:
        src_p, dst_p = src, dst

    degp = _sc_deg(dst, n).T
    out0, g0, norm = _tc_first(degp, feats, W0)
    p1 = _sc_hop(g0, src_p, dst_p)
    out1, g1 = _tc_hop(p1, norm, W1, True)
    p2 = _sc_hop(g1, src_p, dst_p)
    out2, _ = _tc_hop(p2, norm, W2, False)
    return jnp.concatenate([out0, out1, out2], axis=1)


# speedup vs baseline: 10.1395x; 1.1517x over previous
import functools

import jax
import jax.numpy as jnp
from jax import lax
from jax.experimental import pallas as pl
from jax.experimental.pallas import tpu as pltpu
from jax.experimental.pallas import tpu_sc as plsc

NC = 2
NS = 16
CHUNK = 64


def _zero_f32(ref, n_rows, n_cols):
    @pl.loop(0, n_rows)
    def _(i):
        for j in range(n_cols // 16):
            ref[i, pl.ds(j * 16, 16)] = jnp.zeros((16,), jnp.float32)


def _strided_rows(s, n_rows, fn, piece=128):
    n_full = n_rows // piece
    tail = n_rows % piece

    @pl.loop(s, n_full, step=NS)
    def _(p):
        fn(p * piece, piece)

    if tail:
        @pl.when(s == n_full % NS)
        def _():
            fn(n_full * piece, tail)


def _deg_body(dst_hbm, out_hbm, hist, ibuf, n_edges, n_nodes):
    c = lax.axis_index("c")
    s = lax.axis_index("s")
    wid = c * NS + s
    per_tile = n_edges // (NC * NS)

    @pl.loop(0, n_nodes // 16)
    def _(i):
        hist[pl.ds(i * 16, 16)] = jnp.zeros((16,), jnp.float32)

    pltpu.sync_copy(dst_hbm.at[pl.ds(wid * per_tile, per_tile)], ibuf)

    @pl.loop(0, per_tile // 16)
    def _(i):
        v = ibuf[pl.ds(i * 16, 16)]
        cnt, last = plsc.scan_count(v)
        plsc.addupdate_scatter(hist, [v], cnt.astype(jnp.float32), mask=last)

    pltpu.sync_copy(hist, out_hbm.at[wid])


NBUF = 4
NGIF = 3
IB = 80
JUNK = 64


def _hop_body(g_hbm, src_hbm, dst_hbm, out_hbm, acc, sidx, didx, rows,
              gsems, ssems, n_edges, n_nodes):
    c = lax.axis_index("c")
    s = lax.axis_index("s")

    _zero_f32(rows.at[0], CHUNK, 128)
    _strided_rows(s, n_nodes, lambda off, sz: pltpu.sync_copy(
        rows.at[0, pl.ds(0, sz)], acc.at[pl.ds(off, sz)]), piece=CHUNK)
    plsc.subcore_barrier()

    half = (n_edges // CHUNK) // NC
    n0 = half // NS
    nbatch = n0 // IB
    start = c * half + s * n0

    def load_batch(b):
        off = (start + b * IB) * CHUNK
        pltpu.sync_copy(src_hbm.at[pl.ds(off, IB * CHUNK)], sidx)
        pltpu.sync_copy(dst_hbm.at[pl.ds(off, IB * CHUNK)], didx)

    def start_gather(j, sl):
        pltpu.async_copy(g_hbm.at[sidx.at[pl.ds(j * CHUNK, CHUNK)]],
                         rows.at[sl], gsems[sl])

    def start_scatter(j, sl):
        pltpu.async_copy(rows.at[sl],
                         acc.at[didx.at[pl.ds(j * CHUNK, CHUNK)]],
                         ssems[sl], add=True)

    def wait_gather(j, sl):
        pltpu.make_async_copy(g_hbm.at[sidx.at[pl.ds(j * CHUNK, CHUNK)]],
                              rows.at[sl], gsems[sl]).wait()

    def wait_scatter(j, sl):
        pltpu.make_async_copy(rows.at[sl],
                              acc.at[didx.at[pl.ds(j * CHUNK, CHUNK)]],
                              ssems[sl]).wait()

    for b in range(nbatch):
        if b > 0:
            for q in range(NBUF):
                wait_scatter(IB - NBUF + q, (IB - NBUF + q) % NBUF)
        load_batch(b)
        for q in range(NGIF):
            start_gather(q, q)
        for j in range(IB):
            sl = j % NBUF
            wait_gather(j, sl)
            start_scatter(j, sl)
            if j + NGIF < IB:
                if j >= 1:
                    wait_scatter(j - 1, (j - 1) % NBUF)
                start_gather(j + NGIF, (j + NGIF) % NBUF)
    for q in range(NBUF):
        wait_scatter(IB - NBUF + q, (IB - NBUF + q) % NBUF)

    plsc.subcore_barrier()
    _strided_rows(s, n_nodes, lambda off, sz: pltpu.sync_copy(
        acc.at[pl.ds(off, sz)], out_hbm.at[c, pl.ds(off, sz)]), piece=CHUNK)


def _sc_deg(dst, n_nodes):
    n_edges = dst.shape[0]
    mesh = plsc.VectorSubcoreMesh(core_axis_name="c", subcore_axis_name="s")
    body = functools.partial(_deg_body, n_edges=n_edges, n_nodes=n_nodes)
    return pl.kernel(
        body,
        out_type=jax.ShapeDtypeStruct((NC * NS, n_nodes), jnp.float32),
        mesh=mesh,
        scratch_types=[
            pltpu.VMEM((n_nodes,), jnp.float32),
            pltpu.VMEM((n_edges // (NC * NS),), jnp.int32),
        ],
        compiler_params=pltpu.CompilerParams(needs_layout_passes=False),
    )(dst)


def _sc_hop(g, src, dst):
    n_nodes, d = g.shape
    n_edges = src.shape[0]
    mesh = plsc.VectorSubcoreMesh(core_axis_name="c", subcore_axis_name="s")
    body = functools.partial(_hop_body, n_edges=n_edges, n_nodes=n_nodes)
    return pl.kernel(
        body,
        out_type=jax.ShapeDtypeStruct((NC, n_nodes, d), jnp.float32),
        mesh=mesh,
        scratch_types=[
            pltpu.VMEM_SHARED((n_nodes + JUNK, d), jnp.float32),
            pltpu.VMEM((IB * CHUNK,), jnp.int32),
            pltpu.VMEM((IB * CHUNK,), jnp.int32),
            pltpu.VMEM((NBUF, CHUNK, d), jnp.float32),
            [pltpu.SemaphoreType.DMA] * NBUF,
            [pltpu.SemaphoreType.DMA] * NBUF,
        ],
    )(g, src, dst)


_BLK = 1000


def _tc0_body(degp_ref, x_ref, w_ref, out_ref, g_ref, norm_ref):
    deg = jnp.sum(degp_ref[...], axis=1)[:, None]
    norm = lax.rsqrt(jnp.maximum(deg, 1.0))
    x = x_ref[...]
    out_ref[...] = lax.dot_general(
        x, w_ref[...], (((1,), (1,)), ((), ())),
        preferred_element_type=jnp.float32)
    g_ref[...] = x * norm
    norm_ref[...] = norm


def _tc_first(degp, feats, w):
    n, d = feats.shape
    grid = (n // _BLK,)
    return pl.pallas_call(
        _tc0_body,
        grid=grid,
        in_specs=[
            pl.BlockSpec((_BLK, NC * NS), lambda i: (i, 0)),
            pl.BlockSpec((_BLK, d), lambda i: (i, 0)),
            pl.BlockSpec((d, d), lambda i: (0, 0)),
        ],
        out_specs=[
            pl.BlockSpec((_BLK, d), lambda i: (i, 0)),
            pl.BlockSpec((_BLK, d), lambda i: (i, 0)),
            pl.BlockSpec((_BLK, 1), lambda i: (i, 0)),
        ],
        out_shape=[
            jax.ShapeDtypeStruct((n, d), jnp.float32),
            jax.ShapeDtypeStruct((n, d), jnp.float32),
            jax.ShapeDtypeStruct((n, 1), jnp.float32),
        ],
    )(degp, feats, w)


def _tc_hop_body(p_ref, norm_ref, w_ref, out_ref, g_ref):
    nrm = norm_ref[...]
    h = (p_ref[0] + p_ref[1]) * nrm
    out_ref[...] = lax.dot_general(
        h, w_ref[...], (((1,), (1,)), ((), ())),
        preferred_element_type=jnp.float32)
    if g_ref is not None:
        g_ref[...] = h * nrm


def _tc_hop(partials, norm, w, want_g):
    _, n, d = partials.shape
    grid = (n // _BLK,)
    out_shape = [jax.ShapeDtypeStruct((n, d), jnp.float32)]
    out_specs = [pl.BlockSpec((_BLK, d), lambda i: (i, 0))]
    if want_g:
        out_shape.append(jax.ShapeDtypeStruct((n, d), jnp.float32))
        out_specs.append(pl.BlockSpec((_BLK, d), lambda i: (i, 0)))
        body = _tc_hop_body
    else:
        def body(p_ref, norm_ref, w_ref, out_ref):
            _tc_hop_body(p_ref, norm_ref, w_ref, out_ref, None)
    res = pl.pallas_call(
        body,
        grid=grid,
        in_specs=[
            pl.BlockSpec((NC, _BLK, d), lambda i: (0, i, 0)),
            pl.BlockSpec((_BLK, 1), lambda i: (i, 0)),
            pl.BlockSpec((d, d), lambda i: (0, 0)),
        ],
        out_specs=out_specs,
        out_shape=out_shape,
    )(partials, norm, w)
    return res if want_g else (res[0], None)


def kernel(feats, edge_index, W0, W1, W2):
    n = feats.shape[0]
    src = edge_index[0]
    dst = edge_index[1]
    e = src.shape[0]

    unit = CHUNK * NC * NS * IB
    e_pad = -(-e // unit) * unit
    pad = e_pad - e
    if pad:
        fill = jnp.arange(pad, dtype=jnp.int32)
        src_p = jnp.concatenate([src, fill % n])
        dst_p = jnp.concatenate([dst, n + fill % JUNK])
    else:
        src_p, dst_p = src, dst

    degp = _sc_deg(dst, n).T
    out0, g0, norm = _tc_first(degp, feats, W0)
    p1 = _sc_hop(g0, src_p, dst_p)
    out1, g1 = _tc_hop(p1, norm, W1, True)
    p2 = _sc_hop(g1, src_p, dst_p)
    out2, _ = _tc_hop(p2, norm, W2, False)
    return jnp.concatenate([out0, out1, out2], axis=1)


# speedup vs baseline: 10.3382x; 1.0196x over previous
import functools

import jax
import jax.numpy as jnp
from jax import lax
from jax.experimental import pallas as pl
from jax.experimental.pallas import tpu as pltpu
from jax.experimental.pallas import tpu_sc as plsc

NC = 2
NS = 16
CHUNK = 64


def _zero_f32(ref, n_rows, n_cols):
    @pl.loop(0, n_rows)
    def _(i):
        for j in range(n_cols // 16):
            ref[i, pl.ds(j * 16, 16)] = jnp.zeros((16,), jnp.float32)


def _strided_rows(s, n_rows, fn, piece=128):
    n_full = n_rows // piece
    tail = n_rows % piece

    @pl.loop(s, n_full, step=NS)
    def _(p):
        fn(p * piece, piece)

    if tail:
        @pl.when(s == n_full % NS)
        def _():
            fn(n_full * piece, tail)


def _deg_body(dst_hbm, out_hbm, hist, ibuf, n_edges, n_nodes):
    c = lax.axis_index("c")
    s = lax.axis_index("s")
    wid = c * NS + s
    per_tile = n_edges // (NC * NS)

    @pl.loop(0, n_nodes // 16)
    def _(i):
        hist[pl.ds(i * 16, 16)] = jnp.zeros((16,), jnp.float32)

    pltpu.sync_copy(dst_hbm.at[pl.ds(wid * per_tile, per_tile)], ibuf)

    @pl.loop(0, per_tile // 16)
    def _(i):
        v = ibuf[pl.ds(i * 16, 16)]
        cnt, last = plsc.scan_count(v)
        plsc.addupdate_scatter(hist, [v], cnt.astype(jnp.float32), mask=last)

    pltpu.sync_copy(hist, out_hbm.at[wid])


NBUF = 4
NGIF = 3
IB = 80
JUNK = 64


def _hop_body(g_hbm, src_hbm, dst_hbm, out_hbm, acc, sidx, didx, rows,
              gsems, ssems, n_edges, n_nodes):
    c = lax.axis_index("c")
    s = lax.axis_index("s")

    _zero_f32(rows.at[0], CHUNK, 128)
    _strided_rows(s, n_nodes, lambda off, sz: pltpu.sync_copy(
        rows.at[0, pl.ds(0, sz)], acc.at[pl.ds(off, sz)]), piece=CHUNK)
    plsc.subcore_barrier()

    half = (n_edges // CHUNK) // NC
    n0 = half // NS
    nbatch = n0 // IB
    start = c * half + s * n0

    def load_batch(b):
        off = (start + b * IB) * CHUNK
        pltpu.sync_copy(src_hbm.at[pl.ds(off, IB * CHUNK)], sidx)
        pltpu.sync_copy(dst_hbm.at[pl.ds(off, IB * CHUNK)], didx)

    def start_gather(j, sl):
        pltpu.async_copy(g_hbm.at[sidx.at[pl.ds(j * CHUNK, CHUNK)]],
                         rows.at[sl], gsems[sl])

    def start_scatter(j, sl):
        pltpu.async_copy(rows.at[sl],
                         acc.at[didx.at[pl.ds(j * CHUNK, CHUNK)]],
                         ssems[sl], add=True)

    def wait_gather(j, sl):
        pltpu.make_async_copy(g_hbm.at[sidx.at[pl.ds(j * CHUNK, CHUNK)]],
                              rows.at[sl], gsems[sl]).wait()

    def wait_scatter(j, sl):
        pltpu.make_async_copy(rows.at[sl],
                              acc.at[didx.at[pl.ds(j * CHUNK, CHUNK)]],
                              ssems[sl]).wait()

    for b in range(nbatch):
        if b > 0:
            for q in range(NBUF):
                wait_scatter(IB - NBUF + q, (IB - NBUF + q) % NBUF)
        load_batch(b)
        for q in range(NGIF):
            start_gather(q, q)
        for j in range(IB):
            sl = j % NBUF
            wait_gather(j, sl)
            start_scatter(j, sl)
            if j + NGIF < IB:
                if j >= 1:
                    wait_scatter(j - 1, (j - 1) % NBUF)
                start_gather(j + NGIF, (j + NGIF) % NBUF)
    for q in range(NBUF):
        wait_scatter(IB - NBUF + q, (IB - NBUF + q) % NBUF)

    plsc.subcore_barrier()
    _strided_rows(s, n_nodes, lambda off, sz: pltpu.sync_copy(
        acc.at[pl.ds(off, sz)], out_hbm.at[c, pl.ds(off, sz)]), piece=CHUNK)


def _sc_deg(dst, n_nodes):
    n_edges = dst.shape[0]
    mesh = plsc.VectorSubcoreMesh(core_axis_name="c", subcore_axis_name="s")
    body = functools.partial(_deg_body, n_edges=n_edges, n_nodes=n_nodes)
    return pl.kernel(
        body,
        out_type=jax.ShapeDtypeStruct((NC * NS, n_nodes), jnp.float32),
        mesh=mesh,
        scratch_types=[
            pltpu.VMEM((n_nodes,), jnp.float32),
            pltpu.VMEM((n_edges // (NC * NS),), jnp.int32),
        ],
        compiler_params=pltpu.CompilerParams(needs_layout_passes=False),
    )(dst)


def _sc_hop(g, src, dst):
    n_nodes, d = g.shape
    n_edges = src.shape[0]
    mesh = plsc.VectorSubcoreMesh(core_axis_name="c", subcore_axis_name="s")
    body = functools.partial(_hop_body, n_edges=n_edges, n_nodes=n_nodes)
    return pl.kernel(
        body,
        out_type=jax.ShapeDtypeStruct((NC, n_nodes, d), jnp.float32),
        mesh=mesh,
        scratch_types=[
            pltpu.VMEM_SHARED((n_nodes + JUNK, d), jnp.float32),
            pltpu.VMEM((IB * CHUNK,), jnp.int32),
            pltpu.VMEM((IB * CHUNK,), jnp.int32),
            pltpu.VMEM((NBUF, CHUNK, d), jnp.float32),
            [pltpu.SemaphoreType.DMA] * NBUF,
            [pltpu.SemaphoreType.DMA] * NBUF,
        ],
    )(g, src, dst)


_BLK = 1000


def _tc0_body(degp_ref, x_ref, g_ref, norm_ref):
    deg = jnp.sum(degp_ref[...], axis=1)[:, None]
    norm = lax.rsqrt(jnp.maximum(deg, 1.0))
    g_ref[...] = x_ref[...] * norm
    norm_ref[...] = norm


def _tc_first(degp, feats):
    n, d = feats.shape
    grid = (n // _BLK,)
    return pl.pallas_call(
        _tc0_body,
        grid=grid,
        in_specs=[
            pl.BlockSpec((_BLK, NC * NS), lambda i: (i, 0)),
            pl.BlockSpec((_BLK, d), lambda i: (i, 0)),
        ],
        out_specs=[
            pl.BlockSpec((_BLK, d), lambda i: (i, 0)),
            pl.BlockSpec((_BLK, 1), lambda i: (i, 0)),
        ],
        out_shape=[
            jax.ShapeDtypeStruct((n, d), jnp.float32),
            jax.ShapeDtypeStruct((n, 1), jnp.float32),
        ],
    )(degp, feats)


def _tc_mid_body(p_ref, norm_ref, g_ref):
    nrm = norm_ref[...]
    g_ref[...] = (p_ref[0] + p_ref[1]) * nrm * nrm


def _tc_mid(partials, norm):
    _, n, d = partials.shape
    grid = (n // _BLK,)
    return pl.pallas_call(
        _tc_mid_body,
        grid=grid,
        in_specs=[
            pl.BlockSpec((NC, _BLK, d), lambda i: (0, i, 0)),
            pl.BlockSpec((_BLK, 1), lambda i: (i, 0)),
        ],
        out_specs=pl.BlockSpec((_BLK, d), lambda i: (i, 0)),
        out_shape=jax.ShapeDtypeStruct((n, d), jnp.float32),
    )(partials, norm)


def _tc_fin_body(x_ref, p1_ref, p2_ref, norm_ref, w0_ref, w1_ref, w2_ref,
                 out_ref):
    nrm = norm_ref[...]
    h1 = (p1_ref[0] + p1_ref[1]) * nrm
    h2 = (p2_ref[0] + p2_ref[1]) * nrm
    ct = (((1,), (1,)), ((), ()))
    d = x_ref.shape[1]
    out_ref[:, 0:d] = lax.dot_general(
        x_ref[...], w0_ref[...], ct, preferred_element_type=jnp.float32)
    out_ref[:, d:2 * d] = lax.dot_general(
        h1, w1_ref[...], ct, preferred_element_type=jnp.float32)
    out_ref[:, 2 * d:3 * d] = lax.dot_general(
        h2, w2_ref[...], ct, preferred_element_type=jnp.float32)


def _tc_final(feats, p1, p2, norm, w0, w1, w2):
    n, d = feats.shape
    grid = (n // _BLK,)
    wspec = pl.BlockSpec((d, d), lambda i: (0, 0))
    return pl.pallas_call(
        _tc_fin_body,
        grid=grid,
        in_specs=[
            pl.BlockSpec((_BLK, d), lambda i: (i, 0)),
            pl.BlockSpec((NC, _BLK, d), lambda i: (0, i, 0)),
            pl.BlockSpec((NC, _BLK, d), lambda i: (0, i, 0)),
            pl.BlockSpec((_BLK, 1), lambda i: (i, 0)),
            wspec, wspec, wspec,
        ],
        out_specs=pl.BlockSpec((_BLK, 3 * d), lambda i: (i, 0)),
        out_shape=jax.ShapeDtypeStruct((n, 3 * d), jnp.float32),
    )(feats, p1, p2, norm, w0, w1, w2)


def kernel(feats, edge_index, W0, W1, W2):
    n = feats.shape[0]
    src = edge_index[0]
    dst = edge_index[1]
    e = src.shape[0]

    unit = CHUNK * NC * NS * IB
    e_pad = -(-e // unit) * unit
    pad = e_pad - e
    if pad:
        fill = jnp.arange(pad, dtype=jnp.int32)
        src_p = jnp.concatenate([src, fill % n])
        dst_p = jnp.concatenate([dst, n + fill % JUNK])
    else:
        src_p, dst_p = src, dst

    degp = _sc_deg(dst, n).T
    g0, norm = _tc_first(degp, feats)
    p1 = _sc_hop(g0, src_p, dst_p)
    g1 = _tc_mid(p1, norm)
    p2 = _sc_hop(g1, src_p, dst_p)
    return _tc_final(feats, p1, p2, norm, W0, W1, W2)


# speedup vs baseline: 10.4528x; 1.0111x over previous
import functools

import jax
import jax.numpy as jnp
from jax import lax
from jax.experimental import pallas as pl
from jax.experimental.pallas import tpu as pltpu
from jax.experimental.pallas import tpu_sc as plsc

NC = 2
NS = 16
CHUNK = 64


def _zero_f32(ref, n_rows, n_cols):
    @pl.loop(0, n_rows)
    def _(i):
        for j in range(n_cols // 16):
            ref[i, pl.ds(j * 16, 16)] = jnp.zeros((16,), jnp.float32)


def _strided_rows(s, n_rows, fn, piece=128):
    n_full = n_rows // piece
    tail = n_rows % piece

    @pl.loop(s, n_full, step=NS)
    def _(p):
        fn(p * piece, piece)

    if tail:
        @pl.when(s == n_full % NS)
        def _():
            fn(n_full * piece, tail)


def _deg_body(dst_hbm, out_hbm, hist, ibuf, n_edges, n_nodes):
    c = lax.axis_index("c")
    s = lax.axis_index("s")
    wid = c * NS + s
    per_tile = n_edges // (NC * NS)

    @pl.loop(0, n_nodes // 16)
    def _(i):
        hist[pl.ds(i * 16, 16)] = jnp.zeros((16,), jnp.float32)

    pltpu.sync_copy(dst_hbm.at[pl.ds(wid * per_tile, per_tile)], ibuf)

    @pl.loop(0, per_tile // 16)
    def _(i):
        v = ibuf[pl.ds(i * 16, 16)]
        cnt, last = plsc.scan_count(v)
        plsc.addupdate_scatter(hist, [v], cnt.astype(jnp.float32), mask=last)

    pltpu.sync_copy(hist, out_hbm.at[wid])


NBUF = 4
NGIF = 3
IB = 80
JUNK = 64


def _hop_body(g_hbm, src_hbm, dst_hbm, out_hbm, acc, sidx, didx, rows,
              gsems, ssems, n_edges, n_nodes):
    c = lax.axis_index("c")
    s = lax.axis_index("s")

    _zero_f32(rows.at[0], CHUNK, 128)
    _strided_rows(s, n_nodes, lambda off, sz: pltpu.sync_copy(
        rows.at[0, pl.ds(0, sz)], acc.at[pl.ds(off, sz)]), piece=CHUNK)
    plsc.subcore_barrier()

    half = (n_edges // CHUNK) // NC
    n0 = half // NS
    nbatch = n0 // IB
    start = c * half + s * n0

    def load_batch(b):
        off = (start + b * IB) * CHUNK
        pltpu.sync_copy(src_hbm.at[pl.ds(off, IB * CHUNK)], sidx)
        pltpu.sync_copy(dst_hbm.at[pl.ds(off, IB * CHUNK)], didx)

    def start_gather(j, sl):
        pltpu.async_copy(g_hbm.at[sidx.at[pl.ds(j * CHUNK, CHUNK)]],
                         rows.at[sl], gsems[sl])

    def start_scatter(j, sl):
        pltpu.async_copy(rows.at[sl],
                         acc.at[didx.at[pl.ds(j * CHUNK, CHUNK)]],
                         ssems[sl], add=True)

    def wait_gather(j, sl):
        pltpu.make_async_copy(g_hbm.at[sidx.at[pl.ds(j * CHUNK, CHUNK)]],
                              rows.at[sl], gsems[sl]).wait()

    def wait_scatter(j, sl):
        pltpu.make_async_copy(rows.at[sl],
                              acc.at[didx.at[pl.ds(j * CHUNK, CHUNK)]],
                              ssems[sl]).wait()

    for b in range(nbatch):
        if b > 0:
            for q in range(NBUF):
                wait_scatter(IB - NBUF + q, (IB - NBUF + q) % NBUF)
        load_batch(b)
        for q in range(NGIF):
            start_gather(q, q)
        for j in range(IB):
            sl = j % NBUF
            wait_gather(j, sl)
            start_scatter(j, sl)
            if j + NGIF < IB:
                if j >= 1:
                    wait_scatter(j - 1, (j - 1) % NBUF)
                start_gather(j + NGIF, (j + NGIF) % NBUF)
    for q in range(NBUF):
        wait_scatter(IB - NBUF + q, (IB - NBUF + q) % NBUF)

    plsc.subcore_barrier()
    _strided_rows(s, n_nodes, lambda off, sz: pltpu.sync_copy(
        acc.at[pl.ds(off, sz)], out_hbm.at[c, pl.ds(off, sz)]), piece=CHUNK)


def _sc_deg(dst, n_nodes):
    n_edges = dst.shape[0]
    mesh = plsc.VectorSubcoreMesh(core_axis_name="c", subcore_axis_name="s")
    body = functools.partial(_deg_body, n_edges=n_edges, n_nodes=n_nodes)
    return pl.kernel(
        body,
        out_type=jax.ShapeDtypeStruct((NC * NS, n_nodes), jnp.float32),
        mesh=mesh,
        scratch_types=[
            pltpu.VMEM((n_nodes,), jnp.float32),
            pltpu.VMEM((n_edges // (NC * NS),), jnp.int32),
        ],
        compiler_params=pltpu.CompilerParams(needs_layout_passes=False),
    )(dst)


def _sc_hop(g, src, dst):
    n_nodes, d = g.shape
    n_edges = src.shape[0]
    mesh = plsc.VectorSubcoreMesh(core_axis_name="c", subcore_axis_name="s")
    body = functools.partial(_hop_body, n_edges=n_edges, n_nodes=n_nodes)
    return pl.kernel(
        body,
        out_type=jax.ShapeDtypeStruct((NC, n_nodes, d), jnp.float32),
        mesh=mesh,
        scratch_types=[
            pltpu.VMEM_SHARED((n_nodes + JUNK, d), jnp.float32),
            pltpu.VMEM((IB * CHUNK,), jnp.int32),
            pltpu.VMEM((IB * CHUNK,), jnp.int32),
            pltpu.VMEM((NBUF, CHUNK, d), jnp.float32),
            [pltpu.SemaphoreType.DMA] * NBUF,
            [pltpu.SemaphoreType.DMA] * NBUF,
        ],
    )(g, src, dst)


_BLK = 1000


def _tc0_body(degp_ref, x_ref, g_ref, norm_ref):
    deg = jnp.sum(degp_ref[...], axis=1)[:, None]
    norm = lax.rsqrt(jnp.maximum(deg, 1.0))
    g_ref[...] = x_ref[...] * norm
    norm_ref[...] = norm


def _tc_first(degp, feats):
    n, d = feats.shape
    grid = (n // _BLK,)
    return pl.pallas_call(
        _tc0_body,
        grid=grid,
        in_specs=[
            pl.BlockSpec((_BLK, NC * NS), lambda i: (i, 0)),
            pl.BlockSpec((_BLK, d), lambda i: (i, 0)),
        ],
        out_specs=[
            pl.BlockSpec((_BLK, d), lambda i: (i, 0)),
            pl.BlockSpec((_BLK, 1), lambda i: (i, 0)),
        ],
        out_shape=[
            jax.ShapeDtypeStruct((n, d), jnp.float32),
            jax.ShapeDtypeStruct((n, 1), jnp.float32),
        ],
    )(degp, feats)


def _tc_mid_body(p_ref, norm_ref, g_ref):
    nrm = norm_ref[...]
    g_ref[...] = (p_ref[0] + p_ref[1]) * nrm * nrm


def _tc_mid(partials, norm):
    _, n, d = partials.shape
    grid = (n // _BLK,)
    return pl.pallas_call(
        _tc_mid_body,
        grid=grid,
        in_specs=[
            pl.BlockSpec((NC, _BLK, d), lambda i: (0, i, 0)),
            pl.BlockSpec((_BLK, 1), lambda i: (i, 0)),
        ],
        out_specs=pl.BlockSpec((_BLK, d), lambda i: (i, 0)),
        out_shape=jax.ShapeDtypeStruct((n, d), jnp.float32),
    )(partials, norm)


_CT = (((1,), (1,)), ((), ()))


def _tc_lin0(feats, w0):
    n, d = feats.shape

    def body(x_ref, w_ref, out_ref):
        out_ref[...] = lax.dot_general(x_ref[...], w_ref[...], _CT,
                                       preferred_element_type=jnp.float32)

    return pl.pallas_call(
        body,
        grid=(n // _BLK,),
        in_specs=[
            pl.BlockSpec((_BLK, d), lambda i: (i, 0)),
            pl.BlockSpec((d, d), lambda i: (0, 0)),
        ],
        out_specs=pl.BlockSpec((_BLK, d), lambda i: (i, 0)),
        out_shape=jax.ShapeDtypeStruct((n, 3 * d), jnp.float32),
    )(feats, w0)


def _tc_lin(partials, norm, w, buf, col):
    _, n, d = partials.shape

    def body(p_ref, norm_ref, w_ref, buf_ref, out_ref):
        del buf_ref
        h = (p_ref[0] + p_ref[1]) * norm_ref[...]
        out_ref[...] = lax.dot_general(h, w_ref[...], _CT,
                                       preferred_element_type=jnp.float32)

    return pl.pallas_call(
        body,
        grid=(n // _BLK,),
        in_specs=[
            pl.BlockSpec((NC, _BLK, d), lambda i: (0, i, 0)),
            pl.BlockSpec((_BLK, 1), lambda i: (i, 0)),
            pl.BlockSpec((d, d), lambda i: (0, 0)),
            pl.BlockSpec(memory_space=pltpu.MemorySpace.HBM),
        ],
        out_specs=pl.BlockSpec((_BLK, d), lambda i: (i, col)),
        out_shape=jax.ShapeDtypeStruct((n, 3 * d), jnp.float32),
        input_output_aliases={3: 0},
    )(partials, norm, w, buf)


def kernel(feats, edge_index, W0, W1, W2):
    n = feats.shape[0]
    src = edge_index[0]
    dst = edge_index[1]
    e = src.shape[0]

    unit = CHUNK * NC * NS * IB
    e_pad = -(-e // unit) * unit
    pad = e_pad - e
    if pad:
        fill = jnp.arange(pad, dtype=jnp.int32)
        src_p = jnp.concatenate([src, fill % n])
        dst_p = jnp.concatenate([dst, n + fill % JUNK])
    else:
        src_p, dst_p = src, dst

    degp = _sc_deg(dst, n).T
    g0, norm = _tc_first(degp, feats)
    buf = _tc_lin0(feats, W0)
    p1 = _sc_hop(g0, src_p, dst_p)
    g1 = _tc_mid(p1, norm)
    buf = _tc_lin(p1, norm, W1, buf, 1)
    p2 = _sc_hop(g1, src_p, dst_p)
    return _tc_lin(p2, norm, W2, buf, 2)
